# baseline (speedup 1.0000x reference)
import functools

import jax
import jax.numpy as jnp
from jax import lax
from jax.experimental import pallas as pl
from jax.experimental.pallas import tpu as pltpu, tpu_sc as plsc

N = 10000
E = 640000
DP = 128
SDP = 112
K = 128
NCH = 5120
NC = 160
TRASH = N
RPW = 624
RPW_LAST = N - 15 * RPW


_DEPTH = 4


def _scatter_body(y_hbm, src_hbm, dst_hbm, zeros_hbm, out_hbm, *bufs):
    srcb = bufs[0:4]
    dstb = bufs[4:8]
    rows = bufs[8:12]
    agg_sh = bufs[12]
    ss = bufs[13:17]
    sd = bufs[17:21]
    sg = bufs[21:25]
    sid = lax.axis_index("s")
    cbase = sid * NC

    def icopy(hbm, c, buf, sem):
        pltpu.async_copy(hbm.at[cbase + c], buf, sem)

    def iwait(hbm, buf, sem):
        pltpu.make_async_copy(hbm.at[cbase], buf, sem).wait()

    def gather(buf, rows, sem):
        pltpu.async_copy(y_hbm.at[buf], rows, sem)

    def gather_wait(buf, rows, sem):
        pltpu.make_async_copy(y_hbm.at[buf], rows, sem).wait()

    def run(nc):
        def slot(c, j, refill):
            gather_wait(srcb[j], rows[j], sg[j])
            if refill:
                icopy(src_hbm, c + _DEPTH, srcb[j], ss[j])
            iwait(dst_hbm, dstb[j], sd[j])
            pltpu.sync_copy(rows[j], agg_sh.at[dstb[j]], add=True)
            if refill:
                icopy(dst_hbm, c + _DEPTH, dstb[j], sd[j])
                iwait(src_hbm, srcb[j], ss[j])
                gather(srcb[j], rows[j], sg[j])

        for j in range(_DEPTH):
            icopy(src_hbm, j, srcb[j], ss[j])
            icopy(dst_hbm, j, dstb[j], sd[j])
        for j in range(_DEPTH):
            iwait(src_hbm, srcb[j], ss[j])
            gather(srcb[j], rows[j], sg[j])

        @pl.when(sid < 15)
        def _():
            pltpu.sync_copy(zeros_hbm.at[pl.ds(0, RPW)],
                            agg_sh.at[pl.ds(sid * RPW, RPW)])

        @pl.when(sid == 15)
        def _():
            pltpu.sync_copy(zeros_hbm, agg_sh.at[pl.ds(15 * RPW, RPW_LAST)])

        plsc.subcore_barrier()

        def quad(q, carry):
            c0 = _DEPTH * q
            for j in range(_DEPTH):
                slot(c0 + j, j, True)
            return carry

        lax.fori_loop(0, (nc - _DEPTH) // _DEPTH, quad, 0)
        for j in range(_DEPTH):
            slot(nc - _DEPTH + j, j, False)

    run(NC)

    plsc.subcore_barrier()

    @pl.when(sid < 15)
    def _():
        pltpu.sync_copy(agg_sh.at[pl.ds(sid * RPW, RPW)],
                        out_hbm.at[pl.ds(sid * RPW, RPW)])

    @pl.when(sid == 15)
    def _():
        pltpu.sync_copy(agg_sh.at[pl.ds(15 * RPW, RPW_LAST)],
                        out_hbm.at[pl.ds(15 * RPW, RPW_LAST)])


_sc_scatter = functools.partial(
    pl.kernel,
    out_type=jax.ShapeDtypeStruct((N, SDP), jnp.float32),
    mesh=plsc.VectorSubcoreMesh(core_axis_name="c", subcore_axis_name="s",
                                num_cores=1),
    compiler_params=pltpu.CompilerParams(use_tc_tiling_on_sc=False),
    scratch_types=(
        [pltpu.VMEM((K,), jnp.int32)] * (2 * _DEPTH)
        + [pltpu.VMEM((K, SDP), jnp.float32)] * _DEPTH
        + [pltpu.VMEM_SHARED((N + 16, SDP), jnp.float32)]
        + [pltpu.SemaphoreType.DMA] * (3 * _DEPTH)
    ),
)(_scatter_body)


def _edge_agg(y, src2, dst2, zeros):
    h = NCH // 2
    pa = _sc_scatter(y, src2[:h], dst2[:h], zeros)
    pb = _sc_scatter(y, src2[h:], dst2[h:], zeros)
    return pa, pb


_BLK = 2000
_GRID = N // _BLK
_HI = jax.lax.Precision.DEFAULT


def _first_body(nf_ref, w0_ref, b0_ref, w1_ref, b1_ref, o_ref):
    x = jnp.dot(nf_ref[...], w0_ref[...], precision=_HI) + b0_ref[...]
    o_ref[...] = jnp.maximum(
        jnp.dot(x, w1_ref[...], precision=_HI) + b1_ref[...], 0.0)


def _first_layer(nf, w0, b0, w1, b1):
    return pl.pallas_call(
        _first_body,
        grid=(_GRID,),
        in_specs=[
            pl.BlockSpec((_BLK, DP), lambda i: (i, 0)),
            pl.BlockSpec((DP, DP), lambda i: (0, 0)),
            pl.BlockSpec((1, DP), lambda i: (0, 0)),
            pl.BlockSpec((DP, SDP), lambda i: (0, 0)),
            pl.BlockSpec((1, SDP), lambda i: (0, 0)),
        ],
        out_specs=pl.BlockSpec((_BLK, SDP), lambda i: (i, 0)),
        out_shape=jax.ShapeDtypeStruct((N, SDP), jnp.float32),
    )(nf, w0, b0, w1, b1)


def _combine_body(pa_ref, pb_ref, wf_ref, bf_ref, wm_ref, bm_ref, o_ref):
    agg = pa_ref[...] + pb_ref[...]
    x = jnp.maximum(jnp.dot(agg, wf_ref[...], precision=_HI) + bf_ref[...], 0.0)
    o_ref[...] = jnp.maximum(
        jnp.dot(x, wm_ref[...], precision=_HI) + bm_ref[...], 0.0)


def _combine_layer(pa, pb, wf, bf, wm, bm):
    return pl.pallas_call(
        _combine_body,
        grid=(_GRID,),
        in_specs=[
            pl.BlockSpec((_BLK, SDP), lambda i: (i, 0)),
            pl.BlockSpec((_BLK, SDP), lambda i: (i, 0)),
            pl.BlockSpec((SDP, DP), lambda i: (0, 0)),
            pl.BlockSpec((1, DP), lambda i: (0, 0)),
            pl.BlockSpec((DP, SDP), lambda i: (0, 0)),
            pl.BlockSpec((1, SDP), lambda i: (0, 0)),
        ],
        out_specs=pl.BlockSpec((_BLK, SDP), lambda i: (i, 0)),
        out_shape=jax.ShapeDtypeStruct((N, SDP), jnp.float32),
    )(pa, pb, wf, bf, wm, bm)


def _final_body(pa_ref, pb_ref, wf_ref, bf_ref, wro_ref, bro_ref, o_ref):
    agg = pa_ref[...] + pb_ref[...]
    x = jnp.maximum(jnp.dot(agg, wf_ref[...], precision=_HI) + bf_ref[...], 0.0)
    part = jnp.dot(x, wro_ref[...], precision=_HI)
    psum = jnp.sum(part, axis=0, keepdims=True)

    @pl.when(pl.program_id(0) == 0)
    def _():
        o_ref[...] = bro_ref[...]

    o_ref[...] += psum


def _final_layer(pa, pb, wf, bf, wro, bro_scaled):
    return pl.pallas_call(
        _final_body,
        grid=(_GRID,),
        in_specs=[
            pl.BlockSpec((_BLK, SDP), lambda i: (i, 0)),
            pl.BlockSpec((_BLK, SDP), lambda i: (i, 0)),
            pl.BlockSpec((SDP, DP), lambda i: (0, 0)),
            pl.BlockSpec((1, DP), lambda i: (0, 0)),
            pl.BlockSpec((DP, DP), lambda i: (0, 0)),
            pl.BlockSpec((1, DP), lambda i: (0, 0)),
        ],
        out_specs=pl.BlockSpec((1, DP), lambda i: (0, 0)),
        out_shape=jax.ShapeDtypeStruct((1, DP), jnp.float32),
    )(pa, pb, wf, bf, wro, bro_scaled)


def _pad_w(w, r=DP, c=DP):
    out = jnp.zeros((r, c), jnp.float32)
    return out.at[: w.shape[0], : w.shape[1]].set(w)


def _pad_b(b, c=DP):
    out = jnp.zeros((1, c), jnp.float32)
    return out.at[0, : b.shape[0]].set(b)


def kernel(node_feats, edge_index, W_lift, b_lift, Wm1, bm1, Wf1, bf1,
           Wm2, bm2, Wf2, bf2, Wm3, bm3, Wf3, bf3, W_ro, b_ro):
    src = jnp.pad(edge_index[0].reshape(E // K, K), ((0, NCH - E // K), (0, 0)))
    dst = jnp.pad(edge_index[1].reshape(E // K, K), ((0, NCH - E // K), (0, 0)),
                  constant_values=TRASH)

    w0 = _pad_w(jnp.kron(jnp.eye(20, dtype=jnp.float32), W_lift))
    b0 = _pad_b(jnp.tile(b_lift, 20))
    nf = jnp.pad(node_feats.reshape(N, 40), ((0, 0), (0, DP - 40)))

    zeros = jnp.zeros((RPW_LAST, SDP), jnp.float32)

    y1 = _first_layer(nf, w0, b0, _pad_w(Wm1, DP, SDP), _pad_b(bm1, SDP))
    p1a, p1b = _edge_agg(y1, src, dst, zeros)
    y2 = _combine_layer(p1a, p1b, _pad_w(Wf1, SDP, DP), _pad_b(bf1),
                        _pad_w(Wm2, DP, SDP), _pad_b(bm2, SDP))
    p2a, p2b = _edge_agg(y2, src, dst, zeros)
    y3 = _combine_layer(p2a, p2b, _pad_w(Wf2, SDP, DP), _pad_b(bf2),
                        _pad_w(Wm3, DP, SDP), _pad_b(bm3, SDP))
    p3a, p3b = _edge_agg(y3, src, dst, zeros)
    row = _final_layer(p3a, p3b, _pad_w(Wf3, SDP, DP), _pad_b(bf3),
                       _pad_w(W_ro), _pad_b(b_ro * N))
    return row[0, :2]


# speedup vs baseline: 5.5963x; 5.5963x over previous
import functools

import jax
import jax.numpy as jnp
from jax import lax
from jax.experimental import pallas as pl
from jax.experimental.pallas import tpu as pltpu, tpu_sc as plsc

N = 10000
E = 640000
DP = 128
SDP = 112
NW = 32
K = 128
NCH = E // K
NC_HI = 160
NC_LO = 152
RPW = 624
RPW_LAST = N - 15 * RPW


_DEPTH = 4


def _scatter_body(y_hbm, src_hbm, dst_hbm, zeros_hbm, out_hbm, *bufs):
    srcb = bufs[0:4]
    dstb = bufs[4:8]
    rows = bufs[8:12]
    agg_sh = bufs[12]
    ss = bufs[13:17]
    sd = bufs[17:21]
    sg = bufs[21:25]
    core = lax.axis_index("c")
    sid = lax.axis_index("s")
    g = core * 16 + sid
    cbase = g * NC_LO + jnp.minimum(g, 17) * (NC_HI - NC_LO)

    def icopy(hbm, c, buf, sem):
        pltpu.async_copy(hbm.at[cbase + c], buf, sem)

    def iwait(hbm, buf, sem):
        pltpu.make_async_copy(hbm.at[cbase], buf, sem).wait()

    def gather(buf, rows, sem):
        pltpu.async_copy(y_hbm.at[buf], rows, sem)

    def gather_wait(buf, rows, sem):
        pltpu.make_async_copy(y_hbm.at[buf], rows, sem).wait()

    def run(nc):
        def slot(c, j, refill):
            gather_wait(srcb[j], rows[j], sg[j])
            if refill:
                icopy(src_hbm, c + _DEPTH, srcb[j], ss[j])
            iwait(dst_hbm, dstb[j], sd[j])
            pltpu.sync_copy(rows[j], agg_sh.at[dstb[j]], add=True)
            if refill:
                icopy(dst_hbm, c + _DEPTH, dstb[j], sd[j])
                iwait(src_hbm, srcb[j], ss[j])
                gather(srcb[j], rows[j], sg[j])

        for j in range(_DEPTH):
            icopy(src_hbm, j, srcb[j], ss[j])
            icopy(dst_hbm, j, dstb[j], sd[j])
        for j in range(_DEPTH):
            iwait(src_hbm, srcb[j], ss[j])
            gather(srcb[j], rows[j], sg[j])

        @pl.when(sid < 15)
        def _():
            pltpu.sync_copy(zeros_hbm.at[pl.ds(0, RPW)],
                            agg_sh.at[pl.ds(sid * RPW, RPW)])

        @pl.when(sid == 15)
        def _():
            pltpu.sync_copy(zeros_hbm, agg_sh.at[pl.ds(15 * RPW, RPW_LAST)])

        plsc.subcore_barrier()

        def quad(q, carry):
            c0 = _DEPTH * q
            for j in range(_DEPTH):
                slot(c0 + j, j, True)
            return carry

        lax.fori_loop(0, (nc - _DEPTH) // _DEPTH, quad, 0)
        for j in range(_DEPTH):
            slot(nc - _DEPTH + j, j, False)

    @pl.when(g < 17)
    def _():
        run(NC_HI)

    @pl.when(g >= 17)
    def _():
        run(NC_LO)

    plsc.subcore_barrier()

    @pl.when(sid < 15)
    def _():
        pltpu.sync_copy(agg_sh.at[pl.ds(sid * RPW, RPW)],
                        out_hbm.at[pl.ds(core * N + sid * RPW, RPW)])

    @pl.when(sid == 15)
    def _():
        pltpu.sync_copy(agg_sh.at[pl.ds(15 * RPW, RPW_LAST)],
                        out_hbm.at[pl.ds(core * N + 15 * RPW, RPW_LAST)])


_sc_scatter = functools.partial(
    pl.kernel,
    out_type=jax.ShapeDtypeStruct((2 * N, SDP), jnp.float32),
    mesh=plsc.VectorSubcoreMesh(core_axis_name="c", subcore_axis_name="s"),
    compiler_params=pltpu.CompilerParams(use_tc_tiling_on_sc=False),
    scratch_types=(
        [pltpu.VMEM((K,), jnp.int32)] * (2 * _DEPTH)
        + [pltpu.VMEM((K, SDP), jnp.float32)] * _DEPTH
        + [pltpu.VMEM_SHARED((N, SDP), jnp.float32)]
        + [pltpu.SemaphoreType.DMA] * (3 * _DEPTH)
    ),
)(_scatter_body)


def _edge_agg(y, src2, dst2, zeros):
    return _sc_scatter(y, src2, dst2, zeros)


_BLK = 2000
_GRID = N // _BLK
_HI = jax.lax.Precision.DEFAULT


def _first_body(nf_ref, w0_ref, b0_ref, w1_ref, b1_ref, o_ref):
    x = jnp.dot(nf_ref[...], w0_ref[...], precision=_HI) + b0_ref[...]
    o_ref[...] = jnp.maximum(
        jnp.dot(x, w1_ref[...], precision=_HI) + b1_ref[...], 0.0)


def _first_layer(nf, w0, b0, w1, b1):
    return pl.pallas_call(
        _first_body,
        grid=(_GRID,),
        in_specs=[
            pl.BlockSpec((_BLK, DP), lambda i: (i, 0)),
            pl.BlockSpec((DP, DP), lambda i: (0, 0)),
            pl.BlockSpec((1, DP), lambda i: (0, 0)),
            pl.BlockSpec((DP, SDP), lambda i: (0, 0)),
            pl.BlockSpec((1, SDP), lambda i: (0, 0)),
        ],
        out_specs=pl.BlockSpec((_BLK, SDP), lambda i: (i, 0)),
        out_shape=jax.ShapeDtypeStruct((N, SDP), jnp.float32),
    )(nf, w0, b0, w1, b1)


def _combine_body(pa_ref, pb_ref, wf_ref, bf_ref, wm_ref, bm_ref, o_ref):
    agg = pa_ref[...] + pb_ref[...]
    x = jnp.maximum(jnp.dot(agg, wf_ref[...], precision=_HI) + bf_ref[...], 0.0)
    o_ref[...] = jnp.maximum(
        jnp.dot(x, wm_ref[...], precision=_HI) + bm_ref[...], 0.0)


def _combine_layer(parts, wf, bf, wm, bm):
    return pl.pallas_call(
        _combine_body,
        grid=(_GRID,),
        in_specs=[
            pl.BlockSpec((_BLK, SDP), lambda i: (i, 0)),
            pl.BlockSpec((_BLK, SDP), lambda i: (i + _GRID, 0)),
            pl.BlockSpec((SDP, DP), lambda i: (0, 0)),
            pl.BlockSpec((1, DP), lambda i: (0, 0)),
            pl.BlockSpec((DP, SDP), lambda i: (0, 0)),
            pl.BlockSpec((1, SDP), lambda i: (0, 0)),
        ],
        out_specs=pl.BlockSpec((_BLK, SDP), lambda i: (i, 0)),
        out_shape=jax.ShapeDtypeStruct((N, SDP), jnp.float32),
    )(parts, parts, wf, bf, wm, bm)


def _final_body(pa_ref, pb_ref, wf_ref, bf_ref, wro_ref, bro_ref, o_ref):
    agg = pa_ref[...] + pb_ref[...]
    x = jnp.maximum(jnp.dot(agg, wf_ref[...], precision=_HI) + bf_ref[...], 0.0)
    part = jnp.dot(x, wro_ref[...], precision=_HI)
    psum = jnp.sum(part, axis=0, keepdims=True)

    @pl.when(pl.program_id(0) == 0)
    def _():
        o_ref[...] = bro_ref[...]

    o_ref[...] += psum


def _final_layer(parts, wf, bf, wro, bro_scaled):
    return pl.pallas_call(
        _final_body,
        grid=(_GRID,),
        in_specs=[
            pl.BlockSpec((_BLK, SDP), lambda i: (i, 0)),
            pl.BlockSpec((_BLK, SDP), lambda i: (i + _GRID, 0)),
            pl.BlockSpec((SDP, DP), lambda i: (0, 0)),
            pl.BlockSpec((1, DP), lambda i: (0, 0)),
            pl.BlockSpec((DP, DP), lambda i: (0, 0)),
            pl.BlockSpec((1, DP), lambda i: (0, 0)),
        ],
        out_specs=pl.BlockSpec((1, DP), lambda i: (0, 0)),
        out_shape=jax.ShapeDtypeStruct((1, DP), jnp.float32),
    )(parts, parts, wf, bf, wro, bro_scaled)


def _pad_w(w, r=DP, c=DP):
    out = jnp.zeros((r, c), jnp.float32)
    return out.at[: w.shape[0], : w.shape[1]].set(w)


def _pad_b(b, c=DP):
    out = jnp.zeros((1, c), jnp.float32)
    return out.at[0, : b.shape[0]].set(b)


def kernel(node_feats, edge_index, W_lift, b_lift, Wm1, bm1, Wf1, bf1,
           Wm2, bm2, Wf2, bf2, Wm3, bm3, Wf3, bf3, W_ro, b_ro):
    src = edge_index[0].reshape(NCH, K)
    dst = edge_index[1].reshape(NCH, K)

    w0 = _pad_w(jnp.kron(jnp.eye(20, dtype=jnp.float32), W_lift))
    b0 = _pad_b(jnp.tile(b_lift, 20))
    nf = jnp.pad(node_feats.reshape(N, 40), ((0, 0), (0, DP - 40)))

    zeros = jnp.zeros((RPW_LAST, SDP), jnp.float32)

    y1 = _first_layer(nf, w0, b0, _pad_w(Wm1, DP, SDP), _pad_b(bm1, SDP))
    p1 = _edge_agg(y1, src, dst, zeros)
    y2 = _combine_layer(p1, _pad_w(Wf1, SDP, DP), _pad_b(bf1),
                        _pad_w(Wm2, DP, SDP), _pad_b(bm2, SDP))
    p2 = _edge_agg(y2, src, dst, zeros)
    y3 = _combine_layer(p2, _pad_w(Wf2, SDP, DP), _pad_b(bf2),
                        _pad_w(Wm3, DP, SDP), _pad_b(bm3, SDP))
    p3 = _edge_agg(y3, src, dst, zeros)
    row = _final_layer(p3, _pad_w(Wf3, SDP, DP), _pad_b(bf3), _pad_w(W_ro),
                       _pad_b(b_ro * N))
    return row[0, :2]
